# Initial kernel scaffold; baseline (speedup 1.0000x reference)
import jax
import jax.numpy as jnp
from jax.experimental import pallas as pl


def kernel(x, adj, W1, b1, W2, b2):
    raise NotImplementedError("write your pallas kernel here")


# speedup vs baseline: 1.0214x; 1.0214x over previous
import functools

import jax
import jax.numpy as jnp
from jax.experimental import pallas as pl

B, N, F, H = 2, 4096, 128, 128
BM = 512


def _proj_kernel(x_ref, w_ref, o_ref):
    o_ref[...] = jnp.dot(
        x_ref[0], w_ref[...], preferred_element_type=jnp.float32
    )[None]


def _layer1_kernel(adj_ref, s_ref, b_ref, w2_ref, s2_ref, gh_ref):
    i = pl.program_id(1)
    t = jnp.dot(adj_ref[0], s_ref[0], preferred_element_type=jnp.float32)
    h = jnp.maximum(t + b_ref[...], 0.0)
    gh_part = jnp.sum(h, axis=0, keepdims=True)[None]

    @pl.when(i == 0)
    def _():
        gh_ref[...] = gh_part

    @pl.when(i != 0)
    def _():
        gh_ref[...] += gh_part

    s2_ref[...] = jnp.dot(h, w2_ref[...], preferred_element_type=jnp.float32)[None]


def _layer2_kernel(adj_ref, s_ref, b_ref, h_ref, gh_ref):
    i = pl.program_id(1)
    t = jnp.dot(adj_ref[0], s_ref[0], preferred_element_type=jnp.float32)
    h = jnp.maximum(t + b_ref[...], 0.0)
    gh_part = jnp.sum(h, axis=0, keepdims=True)[None]

    @pl.when(i == 0)
    def _():
        gh_ref[...] = gh_part

    @pl.when(i != 0)
    def _():
        gh_ref[...] += gh_part

    h_ref[...] = h[None]


@functools.partial(jax.jit, static_argnames=("interpret",))
def _encoder(x, adj, W1, b1, W2, b2, interpret=False):
    b1r = b1.reshape(1, H)
    b2r = b2.reshape(1, H)

    s1 = pl.pallas_call(
        _proj_kernel,
        grid=(B,),
        in_specs=[
            pl.BlockSpec((1, N, F), lambda b: (b, 0, 0)),
            pl.BlockSpec((F, H), lambda b: (0, 0)),
        ],
        out_specs=pl.BlockSpec((1, N, H), lambda b: (b, 0, 0)),
        out_shape=jax.ShapeDtypeStruct((B, N, H), jnp.float32),
        interpret=interpret,
    )(x, W1)

    num_i = N // BM
    s2, gh1 = pl.pallas_call(
        _layer1_kernel,
        grid=(B, num_i),
        in_specs=[
            pl.BlockSpec((1, BM, N), lambda b, i: (b, i, 0)),
            pl.BlockSpec((1, N, H), lambda b, i: (b, 0, 0)),
            pl.BlockSpec((1, H), lambda b, i: (0, 0)),
            pl.BlockSpec((H, H), lambda b, i: (0, 0)),
        ],
        out_specs=[
            pl.BlockSpec((1, BM, H), lambda b, i: (b, i, 0)),
            pl.BlockSpec((1, 1, H), lambda b, i: (b, 0, 0)),
        ],
        out_shape=[
            jax.ShapeDtypeStruct((B, N, H), jnp.float32),
            jax.ShapeDtypeStruct((B, 1, H), jnp.float32),
        ],
        interpret=interpret,
    )(adj, s1, b1r, W2)

    h2, gh2 = pl.pallas_call(
        _layer2_kernel,
        grid=(B, num_i),
        in_specs=[
            pl.BlockSpec((1, BM, N), lambda b, i: (b, i, 0)),
            pl.BlockSpec((1, N, H), lambda b, i: (b, 0, 0)),
            pl.BlockSpec((1, H), lambda b, i: (0, 0)),
        ],
        out_specs=[
            pl.BlockSpec((1, BM, H), lambda b, i: (b, i, 0)),
            pl.BlockSpec((1, 1, H), lambda b, i: (b, 0, 0)),
        ],
        out_shape=[
            jax.ShapeDtypeStruct((B, N, H), jnp.float32),
            jax.ShapeDtypeStruct((B, 1, H), jnp.float32),
        ],
        interpret=interpret,
    )(adj, s2, b2r)

    gh = jnp.concatenate([gh1[:, 0, :], gh2[:, 0, :]], axis=-1)
    return h2, gh


def kernel(x, adj, W1, b1, W2, b2):
    return _encoder(x, adj, W1, b1, W2, b2)
